# Initial kernel scaffold; baseline (speedup 1.0000x reference)
import jax
import jax.numpy as jnp
from jax.experimental import pallas as pl


def kernel(x, edge_index, batch, eps, W_first, b_first, W_mlp, b_mlp, W_lin1, b_lin1, W_lin2, b_lin2):
    raise NotImplementedError("write your pallas kernel here")


# speedup vs baseline: 3.2738x; 3.2738x over previous
import functools
from functools import partial

import jax
import jax.numpy as jnp
from jax import lax
from jax.experimental import pallas as pl
from jax.experimental.pallas import tpu as pltpu
from jax.experimental.pallas import tpu_sc as plsc

N = 10000
E = 320000
D = 128
H = 128
OUT = 16
G = 64
L = 3
S = 2

NC = 2
NS = 16
NW = NC * NS
CHUNK = 64
NCHUNK = 160
EPW = NCHUNK * CHUNK
EPAD = NW * EPW
NPAD = 10112
ROWS_PER_TILE = NPAD // NS
TRASH_ROW = N

B_BLK = 1000
NBLK = N // B_BLK


NBUF = 2
IB = 16
NIB = NCHUNK // IB


def _seg_sum_body(h_hbm, col3_hbm, row3_hbm, zeros_hbm, out_hbm,
                  colb0, rowb0, colb1, rowb1, row1_v, rows0, rows1,
                  semg0, semg1, semi0, semi1, acc_sh):
    semg = (semg0, semg1)
    rows = (rows0, rows1)
    c = lax.axis_index("c")
    s = lax.axis_index("s")
    wid = s * NC + c
    me = col3_hbm.at[wid]
    mer = row3_hbm.at[wid]

    pltpu.async_copy(me.at[pl.ds(0, IB)], colb0, semi0)
    pltpu.async_copy(mer.at[pl.ds(0, IB)], rowb0, semi0)
    pltpu.async_copy(me.at[pl.ds(IB, IB)], colb1, semi1)
    pltpu.async_copy(mer.at[pl.ds(IB, IB)], rowb1, semi1)
    pltpu.sync_copy(zeros_hbm,
                    acc_sh.at[pl.ds(s * ROWS_PER_TILE, ROWS_PER_TILE)])
    plsc.subcore_barrier()

    def _block(q, colb, rowb, semi):
        pltpu.make_async_copy(me.at[pl.ds(0, IB)], colb, semi).wait()
        pltpu.make_async_copy(mer.at[pl.ds(0, IB)], rowb, semi).wait()

        def _scatter(jj, b):
            for k in range(CHUNK // 16):
                row1_v[pl.ds(k * 16, 16)] = rowb[jj, pl.ds(k * 16, 16)]
            pltpu.sync_copy(rows[b], acc_sh.at[row1_v], add=True)

        for b in range(NBUF):
            pltpu.async_copy(h_hbm.at[colb.at[b]], rows[b], semg[b])

        def _inner(ii, carry):
            j0 = ii * NBUF
            for b in range(NBUF):
                jj = j0 + b
                pltpu.make_async_copy(h_hbm.at[colb.at[jj]], rows[b],
                                      semg[b]).wait()
                _scatter(jj, b)
                pltpu.async_copy(h_hbm.at[colb.at[jj + NBUF]], rows[b],
                                 semg[b])
            return carry

        lax.fori_loop(0, (IB - NBUF) // NBUF, _inner, 0)
        for b in range(NBUF):
            jj = IB - NBUF + b
            pltpu.make_async_copy(h_hbm.at[colb.at[jj]], rows[b],
                                  semg[b]).wait()
            _scatter(jj, b)

        @pl.when(q + 2 < NIB)
        def _():
            off = pl.multiple_of((q + 2) * IB, IB)
            pltpu.async_copy(me.at[pl.ds(off, IB)], colb, semi)
            pltpu.async_copy(mer.at[pl.ds(off, IB)], rowb, semi)

    def _super(bp, carry):
        _block(2 * bp, colb0, rowb0, semi0)
        _block(2 * bp + 1, colb1, rowb1, semi1)
        return carry

    lax.fori_loop(0, NIB // 2, _super, 0)

    plsc.subcore_barrier()
    pltpu.sync_copy(acc_sh.at[pl.ds(s * ROWS_PER_TILE, ROWS_PER_TILE)],
                    out_hbm.at[c].at[pl.ds(s * ROWS_PER_TILE, ROWS_PER_TILE)])


@functools.cache
def _build_seg_sum():
    mesh = plsc.VectorSubcoreMesh(core_axis_name="c", subcore_axis_name="s",
                                  num_cores=NC, num_subcores=NS)
    return pl.kernel(
        _seg_sum_body,
        out_type=jax.ShapeDtypeStruct((NC, NPAD, H), jnp.float32),
        mesh=mesh,
        scratch_types=[
            pltpu.VMEM((IB, CHUNK), jnp.int32),
            pltpu.VMEM((IB, CHUNK), jnp.int32),
            pltpu.VMEM((IB, CHUNK), jnp.int32),
            pltpu.VMEM((IB, CHUNK), jnp.int32),
            pltpu.VMEM((CHUNK,), jnp.int32),
            pltpu.VMEM((CHUNK, H), jnp.float32),
            pltpu.VMEM((CHUNK, H), jnp.float32),
            pltpu.SemaphoreType.DMA,
            pltpu.SemaphoreType.DMA,
            pltpu.SemaphoreType.DMA,
            pltpu.SemaphoreType.DMA,
            pltpu.VMEM_SHARED((NPAD, H), jnp.float32),
        ],
    )


def _mlp0_body(x_ref, w_ref, b_ref, o_ref):
    t = jnp.dot(x_ref[...], w_ref[...], preferred_element_type=jnp.float32)
    o_ref[...] = jnp.maximum(t + b_ref[...], 0.0)


def _gin_mlp_body(p_ref, h_ref, eps_ref, w1_ref, b1_ref, w2_ref, b2_ref,
                  o_ref):
    t = p_ref[0] + p_ref[1] + (1.0 + eps_ref[0, 0]) * h_ref[...]
    t = jnp.maximum(
        jnp.dot(t, w1_ref[...], preferred_element_type=jnp.float32)
        + b1_ref[...], 0.0)
    o_ref[...] = jnp.maximum(
        jnp.dot(t, w2_ref[...], preferred_element_type=jnp.float32)
        + b2_ref[...], 0.0)


def _pool_head_body(h_ref, seg_ref, w1_ref, b1_ref, w2_ref, b2_ref,
                    o_ref, acc_ref):
    i = pl.program_id(0)

    @pl.when(i == 0)
    def _():
        acc_ref[...] = jnp.zeros_like(acc_ref)

    seg = seg_ref[0, 0, :]
    onehot = (seg[None, :]
              == lax.broadcasted_iota(jnp.int32, (G, B_BLK), 0)
              ).astype(jnp.float32)
    acc_ref[...] += jnp.dot(onehot, h_ref[...],
                            preferred_element_type=jnp.float32)

    @pl.when(i == pl.num_programs(0) - 1)
    def _():
        g = jnp.maximum(
            jnp.dot(acc_ref[...], w1_ref[...],
                    preferred_element_type=jnp.float32) + b1_ref[...], 0.0)
        o = jnp.dot(g, w2_ref[...],
                    preferred_element_type=jnp.float32) + b2_ref[...]
        m = jnp.max(o, axis=-1, keepdims=True)
        e = jnp.exp(o - m)
        o_ref[...] = e / jnp.sum(e, axis=-1, keepdims=True)


def _full(shape):
    return pl.BlockSpec(shape, lambda i: tuple(0 for _ in shape))


_mlp0 = pl.pallas_call(
    _mlp0_body,
    grid=(NBLK,),
    in_specs=[
        pl.BlockSpec((B_BLK, D), lambda i: (i, 0)),
        _full((D, H)),
        _full((1, H)),
    ],
    out_specs=pl.BlockSpec((B_BLK, H), lambda i: (i, 0)),
    out_shape=jax.ShapeDtypeStruct((N, H), jnp.float32),
)

_gin_mlp = pl.pallas_call(
    _gin_mlp_body,
    grid=(NBLK,),
    in_specs=[
        pl.BlockSpec((NC, B_BLK, H), lambda i: (0, i, 0)),
        pl.BlockSpec((B_BLK, H), lambda i: (i, 0)),
        _full((1, 1)),
        _full((H, H)),
        _full((1, H)),
        _full((H, H)),
        _full((1, H)),
    ],
    out_specs=pl.BlockSpec((B_BLK, H), lambda i: (i, 0)),
    out_shape=jax.ShapeDtypeStruct((N, H), jnp.float32),
)

_pool_head = pl.pallas_call(
    _pool_head_body,
    grid=(NBLK,),
    in_specs=[
        pl.BlockSpec((B_BLK, H), lambda i: (i, 0)),
        pl.BlockSpec((1, 1, B_BLK), lambda i: (i, 0, 0)),
        _full((H, H)),
        _full((1, H)),
        _full((H, OUT)),
        _full((1, OUT)),
    ],
    out_specs=_full((G, OUT)),
    out_shape=jax.ShapeDtypeStruct((G, OUT), jnp.float32),
    scratch_shapes=[pltpu.VMEM((G, H), jnp.float32)],
)


def kernel(x, edge_index, batch, eps, W_first, b_first, W_mlp, b_mlp,
           W_lin1, b_lin1, W_lin2, b_lin2):
    pad = EPAD - E
    row = jnp.concatenate(
        [edge_index[0], jnp.full((pad,), TRASH_ROW, jnp.int32)]
    ).reshape(NW, NCHUNK, CHUNK)
    col = jnp.concatenate(
        [edge_index[1], jnp.zeros((pad,), jnp.int32)]
    ).reshape(NW, NCHUNK, CHUNK)
    zeros = jnp.zeros((ROWS_PER_TILE, H), jnp.float32)

    seg_sum = _build_seg_sum()
    h = _mlp0(x, W_first, b_first.reshape(1, H))
    for l in range(L):
        parts = seg_sum(h, col, row, zeros)
        h = _gin_mlp(parts, h, eps[l].reshape(1, 1),
                     W_mlp[l, 0], b_mlp[l, 0].reshape(1, H),
                     W_mlp[l, 1], b_mlp[l, 1].reshape(1, H))
    return _pool_head(h, batch.reshape(NBLK, 1, B_BLK),
                      W_lin1, b_lin1.reshape(1, H),
                      W_lin2, b_lin2.reshape(1, OUT))


# speedup vs baseline: 9.5434x; 2.9151x over previous
import functools
from functools import partial

import jax
import jax.numpy as jnp
from jax import lax
from jax.experimental import pallas as pl
from jax.experimental.pallas import tpu as pltpu
from jax.experimental.pallas import tpu_sc as plsc

N = 10000
E = 320000
D = 128
H = 128
OUT = 16
G = 64
L = 3
S = 2

NC = 2
NS = 16
NW = NC * NS
CHUNK = 80
NCHUNK = 128
EPW = NCHUNK * CHUNK
EPPW = E // NW
PADW = EPW - EPPW
NPAD = 10112
ROWS_PER_TILE = NPAD // NS
TRASH_ROW = N

B_BLK = 1000
NBLK = N // B_BLK


NBUF = 2
IB = 16
NIB = NCHUNK // IB


def _seg_sum_body(h_hbm, col3_hbm, row3_hbm, zeros_hbm, out_hbm,
                  colb0, rowb0, colb1, rowb1, row1_v, rows0, rows1,
                  semg0, semg1, semi0, semi1, acc_sh):
    semg = (semg0, semg1)
    rows = (rows0, rows1)
    c = lax.axis_index("c")
    s = lax.axis_index("s")
    wid = s * NC + c
    me = col3_hbm.at[wid]
    mer = row3_hbm.at[wid]

    pltpu.async_copy(me.at[pl.ds(0, IB)], colb0, semi0)
    pltpu.async_copy(mer.at[pl.ds(0, IB)], rowb0, semi0)
    pltpu.async_copy(me.at[pl.ds(IB, IB)], colb1, semi1)
    pltpu.async_copy(mer.at[pl.ds(IB, IB)], rowb1, semi1)
    pltpu.sync_copy(zeros_hbm,
                    acc_sh.at[pl.ds(s * ROWS_PER_TILE, ROWS_PER_TILE)])
    plsc.subcore_barrier()

    def _block(q, colb, rowb, semi):
        pltpu.make_async_copy(me.at[pl.ds(0, IB)], colb, semi).wait()
        pltpu.make_async_copy(mer.at[pl.ds(0, IB)], rowb, semi).wait()

        def _scatter(jj, b):
            for k in range(CHUNK // 16):
                row1_v[pl.ds(k * 16, 16)] = rowb[jj, pl.ds(k * 16, 16)]
            pltpu.sync_copy(rows[b], acc_sh.at[row1_v], add=True)

        for b in range(NBUF):
            pltpu.async_copy(h_hbm.at[colb.at[b]], rows[b], semg[b])

        def _inner(ii, carry):
            j0 = ii * NBUF
            for b in range(NBUF):
                jj = j0 + b
                pltpu.make_async_copy(h_hbm.at[colb.at[jj]], rows[b],
                                      semg[b]).wait()
                _scatter(jj, b)
                pltpu.async_copy(h_hbm.at[colb.at[jj + NBUF]], rows[b],
                                 semg[b])
            return carry

        lax.fori_loop(0, (IB - NBUF) // NBUF, _inner, 0)
        for b in range(NBUF):
            jj = IB - NBUF + b
            pltpu.make_async_copy(h_hbm.at[colb.at[jj]], rows[b],
                                  semg[b]).wait()
            _scatter(jj, b)

        @pl.when(q + 2 < NIB)
        def _():
            off = pl.multiple_of((q + 2) * IB, IB)
            pltpu.async_copy(me.at[pl.ds(off, IB)], colb, semi)
            pltpu.async_copy(mer.at[pl.ds(off, IB)], rowb, semi)

    def _super(bp, carry):
        _block(2 * bp, colb0, rowb0, semi0)
        _block(2 * bp + 1, colb1, rowb1, semi1)
        return carry

    lax.fori_loop(0, NIB // 2, _super, 0)

    plsc.subcore_barrier()
    pltpu.sync_copy(acc_sh.at[pl.ds(s * ROWS_PER_TILE, ROWS_PER_TILE)],
                    out_hbm.at[c].at[pl.ds(s * ROWS_PER_TILE, ROWS_PER_TILE)])


@functools.cache
def _build_seg_sum():
    mesh = plsc.VectorSubcoreMesh(core_axis_name="c", subcore_axis_name="s",
                                  num_cores=NC, num_subcores=NS)
    return pl.kernel(
        _seg_sum_body,
        out_type=jax.ShapeDtypeStruct((NC, NPAD, H), jnp.float32),
        mesh=mesh,
        scratch_types=[
            pltpu.VMEM((IB, CHUNK), jnp.int32),
            pltpu.VMEM((IB, CHUNK), jnp.int32),
            pltpu.VMEM((IB, CHUNK), jnp.int32),
            pltpu.VMEM((IB, CHUNK), jnp.int32),
            pltpu.VMEM((CHUNK,), jnp.int32),
            pltpu.VMEM((CHUNK, H), jnp.float32),
            pltpu.VMEM((CHUNK, H), jnp.float32),
            pltpu.SemaphoreType.DMA,
            pltpu.SemaphoreType.DMA,
            pltpu.SemaphoreType.DMA,
            pltpu.SemaphoreType.DMA,
            pltpu.VMEM_SHARED((NPAD, H), jnp.float32),
        ],
    )


def _mlp0_body(x_ref, w_ref, b_ref, o_ref):
    t = jnp.dot(x_ref[...], w_ref[...], preferred_element_type=jnp.float32)
    o_ref[...] = jnp.maximum(t + b_ref[...], 0.0)


def _gin_mlp_body(p_ref, h_ref, eps_ref, w1_ref, b1_ref, w2_ref, b2_ref,
                  o_ref):
    t = p_ref[0] + p_ref[1] + (1.0 + eps_ref[0, 0]) * h_ref[...]
    t = jnp.maximum(
        jnp.dot(t, w1_ref[...], preferred_element_type=jnp.float32)
        + b1_ref[...], 0.0)
    o_ref[...] = jnp.maximum(
        jnp.dot(t, w2_ref[...], preferred_element_type=jnp.float32)
        + b2_ref[...], 0.0)


def _pool_head_body(h_ref, seg_ref, w1_ref, b1_ref, w2_ref, b2_ref,
                    o_ref, acc_ref):
    i = pl.program_id(0)

    @pl.when(i == 0)
    def _():
        acc_ref[...] = jnp.zeros_like(acc_ref)

    seg = seg_ref[0, 0, :]
    onehot = (seg[None, :]
              == lax.broadcasted_iota(jnp.int32, (G, B_BLK), 0)
              ).astype(jnp.float32)
    acc_ref[...] += jnp.dot(onehot, h_ref[...],
                            preferred_element_type=jnp.float32)

    @pl.when(i == pl.num_programs(0) - 1)
    def _():
        g = jnp.maximum(
            jnp.dot(acc_ref[...], w1_ref[...],
                    preferred_element_type=jnp.float32) + b1_ref[...], 0.0)
        o = jnp.dot(g, w2_ref[...],
                    preferred_element_type=jnp.float32) + b2_ref[...]
        m = jnp.max(o, axis=-1, keepdims=True)
        e = jnp.exp(o - m)
        o_ref[...] = e / jnp.sum(e, axis=-1, keepdims=True)


def _full(shape):
    return pl.BlockSpec(shape, lambda i: tuple(0 for _ in shape))


_mlp0 = pl.pallas_call(
    _mlp0_body,
    grid=(NBLK,),
    in_specs=[
        pl.BlockSpec((B_BLK, D), lambda i: (i, 0)),
        _full((D, H)),
        _full((1, H)),
    ],
    out_specs=pl.BlockSpec((B_BLK, H), lambda i: (i, 0)),
    out_shape=jax.ShapeDtypeStruct((N, H), jnp.float32),
)

_gin_mlp = pl.pallas_call(
    _gin_mlp_body,
    grid=(NBLK,),
    in_specs=[
        pl.BlockSpec((NC, B_BLK, H), lambda i: (0, i, 0)),
        pl.BlockSpec((B_BLK, H), lambda i: (i, 0)),
        _full((1, 1)),
        _full((H, H)),
        _full((1, H)),
        _full((H, H)),
        _full((1, H)),
    ],
    out_specs=pl.BlockSpec((B_BLK, H), lambda i: (i, 0)),
    out_shape=jax.ShapeDtypeStruct((N, H), jnp.float32),
)

_pool_head = pl.pallas_call(
    _pool_head_body,
    grid=(NBLK,),
    in_specs=[
        pl.BlockSpec((B_BLK, H), lambda i: (i, 0)),
        pl.BlockSpec((1, 1, B_BLK), lambda i: (i, 0, 0)),
        _full((H, H)),
        _full((1, H)),
        _full((H, OUT)),
        _full((1, OUT)),
    ],
    out_specs=_full((G, OUT)),
    out_shape=jax.ShapeDtypeStruct((G, OUT), jnp.float32),
    scratch_shapes=[pltpu.VMEM((G, H), jnp.float32)],
)


def kernel(x, edge_index, batch, eps, W_first, b_first, W_mlp, b_mlp,
           W_lin1, b_lin1, W_lin2, b_lin2):
    pad_rows = TRASH_ROW + (jnp.arange(PADW, dtype=jnp.int32) % (NPAD - N))
    pad_cols = (jnp.arange(PADW, dtype=jnp.int32) * 37) % N
    row = jnp.concatenate(
        [edge_index[0].reshape(NW, EPPW),
         jnp.broadcast_to(pad_rows, (NW, PADW))], axis=1
    ).reshape(NW, NCHUNK, CHUNK)
    col = jnp.concatenate(
        [edge_index[1].reshape(NW, EPPW),
         jnp.broadcast_to(pad_cols, (NW, PADW))], axis=1
    ).reshape(NW, NCHUNK, CHUNK)
    zeros = jnp.zeros((ROWS_PER_TILE, H), jnp.float32)

    seg_sum = _build_seg_sum()
    h = _mlp0(x, W_first, b_first.reshape(1, H))
    for l in range(L):
        parts = seg_sum(h, col, row, zeros)
        h = _gin_mlp(parts, h, eps[l].reshape(1, 1),
                     W_mlp[l, 0], b_mlp[l, 0].reshape(1, H),
                     W_mlp[l, 1], b_mlp[l, 1].reshape(1, H))
    return _pool_head(h, batch.reshape(NBLK, 1, B_BLK),
                      W_lin1, b_lin1.reshape(1, H),
                      W_lin2, b_lin2.reshape(1, OUT))


# speedup vs baseline: 10.1105x; 1.0594x over previous
import functools
from functools import partial

import jax
import jax.numpy as jnp
from jax import lax
from jax.experimental import pallas as pl
from jax.experimental.pallas import tpu as pltpu
from jax.experimental.pallas import tpu_sc as plsc

N = 10000
E = 320000
D = 128
H = 128
OUT = 16
G = 64
L = 3
S = 2

NC = 2
NS = 16
NW = NC * NS
CHUNK = 128
NCHUNK = 80
EPW = NCHUNK * CHUNK
EPPW = E // NW
PADW = EPW - EPPW
NPAD = 10112
ROWS_PER_TILE = NPAD // NS
TRASH_ROW = N

B_BLK = 1000
NBLK = N // B_BLK


NBUF = 2
IB = 8
NIB = NCHUNK // IB


def _seg_sum_body(h_hbm, col3_hbm, row3_hbm, zeros_hbm, out_hbm,
                  colb0, rowb0, colb1, rowb1, row1_v, rows0, rows1,
                  semg0, semg1, semi0, semi1, acc_sh):
    semg = (semg0, semg1)
    rows = (rows0, rows1)
    c = lax.axis_index("c")
    s = lax.axis_index("s")
    wid = s * NC + c
    me = col3_hbm.at[wid]
    mer = row3_hbm.at[wid]

    pltpu.async_copy(me.at[pl.ds(0, IB)], colb0, semi0)
    pltpu.async_copy(mer.at[pl.ds(0, IB)], rowb0, semi0)
    pltpu.async_copy(me.at[pl.ds(IB, IB)], colb1, semi1)
    pltpu.async_copy(mer.at[pl.ds(IB, IB)], rowb1, semi1)
    pltpu.sync_copy(zeros_hbm,
                    acc_sh.at[pl.ds(s * ROWS_PER_TILE, ROWS_PER_TILE)])
    plsc.subcore_barrier()

    def _block(q, colb, rowb, semi):
        pltpu.make_async_copy(me.at[pl.ds(0, IB)], colb, semi).wait()
        pltpu.make_async_copy(mer.at[pl.ds(0, IB)], rowb, semi).wait()

        def _scatter(jj, b):
            for k in range(CHUNK // 16):
                row1_v[pl.ds(k * 16, 16)] = rowb[jj, pl.ds(k * 16, 16)]
            pltpu.sync_copy(rows[b], acc_sh.at[row1_v], add=True)

        for b in range(NBUF):
            pltpu.async_copy(h_hbm.at[colb.at[b]], rows[b], semg[b])

        def _inner(ii, carry):
            j0 = ii * NBUF
            for b in range(NBUF):
                jj = j0 + b
                pltpu.make_async_copy(h_hbm.at[colb.at[jj]], rows[b],
                                      semg[b]).wait()
                _scatter(jj, b)
                pltpu.async_copy(h_hbm.at[colb.at[jj + NBUF]], rows[b],
                                 semg[b])
            return carry

        lax.fori_loop(0, (IB - NBUF) // NBUF, _inner, 0)
        for b in range(NBUF):
            jj = IB - NBUF + b
            pltpu.make_async_copy(h_hbm.at[colb.at[jj]], rows[b],
                                  semg[b]).wait()
            _scatter(jj, b)

        @pl.when(q + 2 < NIB)
        def _():
            off = pl.multiple_of((q + 2) * IB, IB)
            pltpu.async_copy(me.at[pl.ds(off, IB)], colb, semi)
            pltpu.async_copy(mer.at[pl.ds(off, IB)], rowb, semi)

    def _super(bp, carry):
        _block(2 * bp, colb0, rowb0, semi0)
        _block(2 * bp + 1, colb1, rowb1, semi1)
        return carry

    lax.fori_loop(0, NIB // 2, _super, 0)

    plsc.subcore_barrier()
    pltpu.sync_copy(acc_sh.at[pl.ds(s * ROWS_PER_TILE, ROWS_PER_TILE)],
                    out_hbm.at[c].at[pl.ds(s * ROWS_PER_TILE, ROWS_PER_TILE)])


@functools.cache
def _build_seg_sum():
    mesh = plsc.VectorSubcoreMesh(core_axis_name="c", subcore_axis_name="s",
                                  num_cores=NC, num_subcores=NS)
    return pl.kernel(
        _seg_sum_body,
        out_type=jax.ShapeDtypeStruct((NC, NPAD, H), jnp.float32),
        mesh=mesh,
        scratch_types=[
            pltpu.VMEM((IB, CHUNK), jnp.int32),
            pltpu.VMEM((IB, CHUNK), jnp.int32),
            pltpu.VMEM((IB, CHUNK), jnp.int32),
            pltpu.VMEM((IB, CHUNK), jnp.int32),
            pltpu.VMEM((CHUNK,), jnp.int32),
            pltpu.VMEM((CHUNK, H), jnp.float32),
            pltpu.VMEM((CHUNK, H), jnp.float32),
            pltpu.SemaphoreType.DMA,
            pltpu.SemaphoreType.DMA,
            pltpu.SemaphoreType.DMA,
            pltpu.SemaphoreType.DMA,
            pltpu.VMEM_SHARED((NPAD, H), jnp.float32),
        ],
    )


def _mlp0_body(x_ref, w_ref, b_ref, o_ref):
    t = jnp.dot(x_ref[...], w_ref[...], preferred_element_type=jnp.float32)
    o_ref[...] = jnp.maximum(t + b_ref[...], 0.0)


def _gin_mlp_body(p_ref, h_ref, eps_ref, w1_ref, b1_ref, w2_ref, b2_ref,
                  o_ref):
    t = p_ref[0] + p_ref[1] + (1.0 + eps_ref[0, 0]) * h_ref[...]
    t = jnp.maximum(
        jnp.dot(t, w1_ref[...], preferred_element_type=jnp.float32)
        + b1_ref[...], 0.0)
    o_ref[...] = jnp.maximum(
        jnp.dot(t, w2_ref[...], preferred_element_type=jnp.float32)
        + b2_ref[...], 0.0)


def _pool_head_body(h_ref, seg_ref, w1_ref, b1_ref, w2_ref, b2_ref,
                    o_ref, acc_ref):
    i = pl.program_id(0)

    @pl.when(i == 0)
    def _():
        acc_ref[...] = jnp.zeros_like(acc_ref)

    seg = seg_ref[0, 0, :]
    onehot = (seg[None, :]
              == lax.broadcasted_iota(jnp.int32, (G, B_BLK), 0)
              ).astype(jnp.float32)
    acc_ref[...] += jnp.dot(onehot, h_ref[...],
                            preferred_element_type=jnp.float32)

    @pl.when(i == pl.num_programs(0) - 1)
    def _():
        g = jnp.maximum(
            jnp.dot(acc_ref[...], w1_ref[...],
                    preferred_element_type=jnp.float32) + b1_ref[...], 0.0)
        o = jnp.dot(g, w2_ref[...],
                    preferred_element_type=jnp.float32) + b2_ref[...]
        m = jnp.max(o, axis=-1, keepdims=True)
        e = jnp.exp(o - m)
        o_ref[...] = e / jnp.sum(e, axis=-1, keepdims=True)


def _full(shape):
    return pl.BlockSpec(shape, lambda i: tuple(0 for _ in shape))


_mlp0 = pl.pallas_call(
    _mlp0_body,
    grid=(NBLK,),
    in_specs=[
        pl.BlockSpec((B_BLK, D), lambda i: (i, 0)),
        _full((D, H)),
        _full((1, H)),
    ],
    out_specs=pl.BlockSpec((B_BLK, H), lambda i: (i, 0)),
    out_shape=jax.ShapeDtypeStruct((N, H), jnp.float32),
)

_gin_mlp = pl.pallas_call(
    _gin_mlp_body,
    grid=(NBLK,),
    in_specs=[
        pl.BlockSpec((NC, B_BLK, H), lambda i: (0, i, 0)),
        pl.BlockSpec((B_BLK, H), lambda i: (i, 0)),
        _full((1, 1)),
        _full((H, H)),
        _full((1, H)),
        _full((H, H)),
        _full((1, H)),
    ],
    out_specs=pl.BlockSpec((B_BLK, H), lambda i: (i, 0)),
    out_shape=jax.ShapeDtypeStruct((N, H), jnp.float32),
)

_pool_head = pl.pallas_call(
    _pool_head_body,
    grid=(NBLK,),
    in_specs=[
        pl.BlockSpec((B_BLK, H), lambda i: (i, 0)),
        pl.BlockSpec((1, 1, B_BLK), lambda i: (i, 0, 0)),
        _full((H, H)),
        _full((1, H)),
        _full((H, OUT)),
        _full((1, OUT)),
    ],
    out_specs=_full((G, OUT)),
    out_shape=jax.ShapeDtypeStruct((G, OUT), jnp.float32),
    scratch_shapes=[pltpu.VMEM((G, H), jnp.float32)],
)


def kernel(x, edge_index, batch, eps, W_first, b_first, W_mlp, b_mlp,
           W_lin1, b_lin1, W_lin2, b_lin2):
    pad_rows = TRASH_ROW + (jnp.arange(PADW, dtype=jnp.int32) % (NPAD - N))
    pad_cols = (jnp.arange(PADW, dtype=jnp.int32) * 37) % N
    row = jnp.concatenate(
        [edge_index[0].reshape(NW, EPPW),
         jnp.broadcast_to(pad_rows, (NW, PADW))], axis=1
    ).reshape(NW, NCHUNK, CHUNK)
    col = jnp.concatenate(
        [edge_index[1].reshape(NW, EPPW),
         jnp.broadcast_to(pad_cols, (NW, PADW))], axis=1
    ).reshape(NW, NCHUNK, CHUNK)
    zeros = jnp.zeros((ROWS_PER_TILE, H), jnp.float32)

    seg_sum = _build_seg_sum()
    h = _mlp0(x, W_first, b_first.reshape(1, H))
    for l in range(L):
        parts = seg_sum(h, col, row, zeros)
        h = _gin_mlp(parts, h, eps[l].reshape(1, 1),
                     W_mlp[l, 0], b_mlp[l, 0].reshape(1, H),
                     W_mlp[l, 1], b_mlp[l, 1].reshape(1, H))
    return _pool_head(h, batch.reshape(NBLK, 1, B_BLK),
                      W_lin1, b_lin1.reshape(1, H),
                      W_lin2, b_lin2.reshape(1, OUT))
